# baseline (speedup 1.0000x reference)
import jax
import jax.numpy as jnp
from jax import lax
from jax.experimental import pallas as pl
from jax.experimental.pallas import tpu as pltpu
from jax.experimental.pallas import tpu_sc as plsc

_N, _E, _D, _G, _C = 10000, 320000, 128, 64, 10
_BN_INV = (1.0 + 1e-5) ** -0.5
_NC, _NS = 2, 16
_NW = _NC * _NS
_EK = 80
_EPW = _E // _NW
_CPW = _EPW // _EK
_ZR = 400
_NZCH = _N // _ZR


_CAP = 128


def _edge_agg_body(x_hbm, src_hbm, dst_hbm, zeros_hbm, out_hbm,
                   acc, src_v, dst_v, rows_v, stage_r, stage_i, sem):
    cid = lax.axis_index("c")
    sid = lax.axis_index("s")
    w = cid * _NS + sid

    nz = (_NZCH - sid + _NS - 1) // _NS

    def zbody(j, c):
        r = pl.multiple_of((j * _NS + sid) * _ZR, 8)
        pltpu.sync_copy(zeros_hbm.at[pl.ds(r, _ZR)], acc.at[pl.ds(r, _ZR)])
        return c

    lax.fori_loop(0, nz, zbody, 0)

    zi = jnp.zeros((16,), jnp.int32)

    def clear_stage():
        pltpu.sync_copy(zeros_hbm.at[pl.ds(0, _CAP)], stage_r)
        for k in range(_CAP // 16):
            stage_i[pl.ds(k * 16, 16)] = zi

    clear_stage()
    plsc.subcore_barrier()

    acc0 = tuple(jnp.zeros((16,), jnp.float32) for _ in range(8))
    iota16 = lax.iota(jnp.int32, 16)
    lane0 = iota16 == 0

    def chunk(i, carry):
        base = pl.multiple_of(w * _EPW + i * _EK, 8)
        pltpu.sync_copy(src_hbm.at[pl.ds(base, _EK)], src_v)
        pltpu.sync_copy(dst_hbm.at[pl.ds(base, _EK)], dst_v)
        pltpu.async_copy(x_hbm.at[src_v], rows_v, sem).wait()

        def group(g, c):
            prev, cnt, accs = c
            d16 = dst_v[pl.ds(g * 16, 16)]
            for l in range(16):
                d = d16[l]
                new = d != prev
                cnt2 = cnt + jnp.where(new, 1, 0)

                def flush():
                    pltpu.sync_copy(stage_r, acc.at[stage_i], add=True)
                    clear_stage()

                pl.when(cnt2 == _CAP)(flush)
                cnt3 = jnp.where(cnt2 == _CAP, 0, cnt2)

                e = g * 16 + l
                c16 = jnp.full((16,), cnt3, jnp.int32)
                newaccs = []
                for j in range(8):
                    row = rows_v[e, pl.ds(j * 16, 16)]
                    a = jnp.where(new, 0.0, accs[j]) + row
                    stage_r[cnt3, pl.ds(j * 16, 16)] = a
                    newaccs.append(a)
                plsc.store_scatter(stage_i, [c16],
                                   jnp.full((16,), d, jnp.int32), mask=lane0)
                prev, cnt, accs = d, cnt3, tuple(newaccs)
            return (prev, cnt, accs)

        return lax.fori_loop(0, _EK // 16, group, carry)

    lax.fori_loop(0, _CPW, chunk, (jnp.int32(-1), jnp.int32(-1), acc0))
    pltpu.sync_copy(stage_r, acc.at[stage_i], add=True)
    plsc.subcore_barrier()

    def wbody(j, c):
        r = pl.multiple_of((j * _NS + sid) * _ZR, 8)
        pltpu.sync_copy(acc.at[pl.ds(r, _ZR)],
                        out_hbm.at[cid, pl.ds(r, _ZR)])
        return c

    lax.fori_loop(0, nz, wbody, 0)


_edge_agg_cache = []


def _edge_agg(x, src, dst, zeros):
    if not _edge_agg_cache:
        _edge_agg_cache.append(pl.kernel(
            _edge_agg_body,
            out_type=jax.ShapeDtypeStruct((_NC, _N, _D), jnp.float32),
            mesh=plsc.VectorSubcoreMesh(core_axis_name="c",
                                        subcore_axis_name="s",
                                        num_cores=_NC, num_subcores=_NS),
            compiler_params=pltpu.CompilerParams(needs_layout_passes=False),
            scratch_types=[
                pltpu.VMEM_SHARED((_N, _D), jnp.float32),
                pltpu.VMEM((_EK,), jnp.int32),
                pltpu.VMEM((_EK,), jnp.int32),
                pltpu.VMEM((_EK, _D), jnp.float32),
                pltpu.VMEM((_CAP, _D), jnp.float32),
                pltpu.VMEM((_CAP,), jnp.int32),
                pltpu.SemaphoreType.DMA,
            ],
        ))
    return _edge_agg_cache[0](x, src, dst, zeros)


def _layer_body(x_ref, p_ref, w_ref, b_ref, g_ref, be_ref, o_ref):
    z = x_ref[...] + (p_ref[0] + p_ref[1])
    z = jnp.dot(z, w_ref[...], preferred_element_type=jnp.float32)
    z = z + b_ref[...]
    z = (z / jnp.sqrt(1.0 + 1e-5)) * g_ref[...] + be_ref[...]
    o_ref[...] = jnp.maximum(z, 0.0)


_ROW_BLK = 1000


def _layer(x, p, W, b, g, be):
    return pl.pallas_call(
        _layer_body,
        grid=(_N // _ROW_BLK,),
        in_specs=[
            pl.BlockSpec((_ROW_BLK, _D), lambda i: (i, 0)),
            pl.BlockSpec((_NC, _ROW_BLK, _D), lambda i: (0, i, 0)),
            pl.BlockSpec((_D, _D), lambda i: (0, 0)),
            pl.BlockSpec((1, _D), lambda i: (0, 0)),
            pl.BlockSpec((1, _D), lambda i: (0, 0)),
            pl.BlockSpec((1, _D), lambda i: (0, 0)),
        ],
        out_specs=pl.BlockSpec((_ROW_BLK, _D), lambda i: (i, 0)),
        out_shape=jax.ShapeDtypeStruct((_N, _D), jnp.float32),
    )(x, p, W, b.reshape(1, _D), g.reshape(1, _D), be.reshape(1, _D))


def _final_body(x_ref, batch_ref, f1w_ref, f1b_ref, f2w_ref, f2b_ref, o_ref):
    gid = lax.broadcasted_iota(jnp.int32, (_G, _N), 0)
    onehot = (batch_ref[...] == gid).astype(jnp.float32)
    cnt = jnp.sum(onehot, axis=1, keepdims=True)
    sums = jnp.dot(onehot, x_ref[...], preferred_element_type=jnp.float32,
                   precision=lax.Precision.HIGHEST)
    pooled = sums / jnp.maximum(cnt, 1.0)
    z = jnp.dot(pooled, f1w_ref[...], preferred_element_type=jnp.float32)
    z = z + f1b_ref[...]
    z = jnp.where(z > 0.0, z, jnp.exp(jnp.minimum(z, 0.0)) - 1.0)
    z = jnp.dot(z, f2w_ref[...], preferred_element_type=jnp.float32)
    z = z + f2b_ref[...]
    m = jnp.max(z, axis=0, keepdims=True)
    lse = jnp.log(jnp.sum(jnp.exp(z - m), axis=0, keepdims=True))
    o_ref[...] = z - m - lse


def _final(x, batch, fc1W, fc1b, fc2W, fc2b):
    return pl.pallas_call(
        _final_body,
        out_shape=jax.ShapeDtypeStruct((_G, _C), jnp.float32),
    )(x, batch.reshape(1, _N), fc1W, fc1b.reshape(1, _D),
      fc2W, fc2b.reshape(1, _C))


def kernel(h, edge_index, edge_attr, batch,
           W0, b0, g0, be0, W1, b1, g1, be1,
           W2, b2, g2, be2, W3, b3, g3, be3,
           fc1W, fc1b, fc2W, fc2b):
    del edge_attr
    zeros = jnp.zeros((_N, _D), jnp.float32)
    perm = jnp.argsort(edge_index[1], stable=True)
    src = edge_index[0][perm]
    dst = edge_index[1][perm]
    x = h
    for (W, b, g, be) in ((W0, b0, g0, be0), (W1, b1, g1, be1),
                          (W2, b2, g2, be2), (W3, b3, g3, be3)):
        p = _edge_agg(x, src, dst, zeros)
        x = _layer(x, p, W, b, g, be)
    return _final(x, batch, fc1W, fc1b, fc2W, fc2b)


# speedup vs baseline: 1.1455x; 1.1455x over previous
import jax
import jax.numpy as jnp
from jax import lax
from jax.experimental import pallas as pl
from jax.experimental.pallas import tpu as pltpu
from jax.experimental.pallas import tpu_sc as plsc

_N, _E, _D, _G, _C = 10000, 320000, 128, 64, 10
_BN_INV = (1.0 + 1e-5) ** -0.5
_NC, _NS = 2, 16
_NW = _NC * _NS
_EK = 80
_EPW = _E // _NW
_CPW = _EPW // _EK
_ZR = 400
_NZCH = _N // _ZR


_CAP = 128


def _edge_agg_body(x_hbm, src_hbm, dst_hbm, zeros_hbm, out_hbm,
                   acc, src_v, dst_v, rows_v, src_v2, dst_v2, rows_v2,
                   stage_r, stage_i, sem, sem2):
    cid = lax.axis_index("c")
    sid = lax.axis_index("s")
    w = cid * _NS + sid

    nz = (_NZCH - sid + _NS - 1) // _NS

    def zbody(j, c):
        r = pl.multiple_of((j * _NS + sid) * _ZR, 8)
        pltpu.sync_copy(zeros_hbm.at[pl.ds(r, _ZR)], acc.at[pl.ds(r, _ZR)])
        return c

    lax.fori_loop(0, nz, zbody, 0)

    zi = jnp.zeros((16,), jnp.int32)

    def clear_stage():
        pltpu.sync_copy(zeros_hbm.at[pl.ds(0, _CAP)], stage_r)
        for k in range(_CAP // 16):
            stage_i[pl.ds(k * 16, 16)] = zi

    clear_stage()
    plsc.subcore_barrier()

    acc0 = tuple(jnp.zeros((16,), jnp.float32) for _ in range(8))
    iota16 = lax.iota(jnp.int32, 16)
    lane0 = iota16 == 0

    src_vs = (src_v, src_v2)
    dst_vs = (dst_v, dst_v2)
    rows_vs = (rows_v, rows_v2)
    sems = (sem, sem2)

    def start(i, b):
        base = pl.multiple_of(w * _EPW + i * _EK, 8)
        pltpu.sync_copy(src_hbm.at[pl.ds(base, _EK)], src_vs[b])
        pltpu.sync_copy(dst_hbm.at[pl.ds(base, _EK)], dst_vs[b])
        pltpu.async_copy(x_hbm.at[src_vs[b]], rows_vs[b], sems[b])

    def process(b, carry):
        def group(g, c):
            prev, cnt, accs = c
            d16 = dst_vs[b][pl.ds(g * 16, 16)]
            for l in range(16):
                d = d16[l]
                new = d != prev
                cnt2 = cnt + jnp.where(new, 1, 0)

                def flush():
                    pltpu.sync_copy(stage_r, acc.at[stage_i], add=True)
                    clear_stage()

                pl.when(cnt2 == _CAP)(flush)
                cnt3 = jnp.where(cnt2 == _CAP, 0, cnt2)

                e = g * 16 + l
                c16 = jnp.full((16,), cnt3, jnp.int32)
                newaccs = []
                for j in range(8):
                    row = rows_vs[b][e, pl.ds(j * 16, 16)]
                    a = jnp.where(new, 0.0, accs[j]) + row
                    stage_r[cnt3, pl.ds(j * 16, 16)] = a
                    newaccs.append(a)
                plsc.store_scatter(stage_i, [c16],
                                   jnp.full((16,), d, jnp.int32), mask=lane0)
                prev, cnt, accs = d, cnt3, tuple(newaccs)
            return (prev, cnt, accs)

        return lax.fori_loop(0, _EK // 16, group, carry)

    start(0, 0)

    def pair(p, carry):
        for b in range(2):
            i = p * 2 + b
            pltpu.make_async_copy(x_hbm.at[src_vs[b]], rows_vs[b],
                                  sems[b]).wait()
            start(i + 1, 1 - b)
            carry = process(b, carry)
        return carry

    carry = lax.fori_loop(0, (_CPW - 1) // 2, pair,
                          (jnp.int32(-1), jnp.int32(-1), acc0))
    pltpu.make_async_copy(x_hbm.at[src_vs[0]], rows_vs[0], sems[0]).wait()
    process(0, carry)
    pltpu.sync_copy(stage_r, acc.at[stage_i], add=True)
    plsc.subcore_barrier()

    def wbody(j, c):
        r = pl.multiple_of((j * _NS + sid) * _ZR, 8)
        pltpu.sync_copy(acc.at[pl.ds(r, _ZR)],
                        out_hbm.at[cid, pl.ds(r, _ZR)])
        return c

    lax.fori_loop(0, nz, wbody, 0)


_edge_agg_cache = []


def _edge_agg(x, src, dst, zeros):
    if not _edge_agg_cache:
        _edge_agg_cache.append(pl.kernel(
            _edge_agg_body,
            out_type=jax.ShapeDtypeStruct((_NC, _N, _D), jnp.float32),
            mesh=plsc.VectorSubcoreMesh(core_axis_name="c",
                                        subcore_axis_name="s",
                                        num_cores=_NC, num_subcores=_NS),
            compiler_params=pltpu.CompilerParams(needs_layout_passes=False),
            scratch_types=[
                pltpu.VMEM_SHARED((_N, _D), jnp.float32),
                pltpu.VMEM((_EK,), jnp.int32),
                pltpu.VMEM((_EK,), jnp.int32),
                pltpu.VMEM((_EK, _D), jnp.float32),
                pltpu.VMEM((_EK,), jnp.int32),
                pltpu.VMEM((_EK,), jnp.int32),
                pltpu.VMEM((_EK, _D), jnp.float32),
                pltpu.VMEM((_CAP, _D), jnp.float32),
                pltpu.VMEM((_CAP,), jnp.int32),
                pltpu.SemaphoreType.DMA,
                pltpu.SemaphoreType.DMA,
            ],
        ))
    return _edge_agg_cache[0](x, src, dst, zeros)


def _layer_body(x_ref, p_ref, w_ref, b_ref, g_ref, be_ref, o_ref):
    z = x_ref[...] + (p_ref[0] + p_ref[1])
    z = jnp.dot(z, w_ref[...], preferred_element_type=jnp.float32)
    z = z + b_ref[...]
    z = (z / jnp.sqrt(1.0 + 1e-5)) * g_ref[...] + be_ref[...]
    o_ref[...] = jnp.maximum(z, 0.0)


_ROW_BLK = 1000


def _layer(x, p, W, b, g, be):
    return pl.pallas_call(
        _layer_body,
        grid=(_N // _ROW_BLK,),
        in_specs=[
            pl.BlockSpec((_ROW_BLK, _D), lambda i: (i, 0)),
            pl.BlockSpec((_NC, _ROW_BLK, _D), lambda i: (0, i, 0)),
            pl.BlockSpec((_D, _D), lambda i: (0, 0)),
            pl.BlockSpec((1, _D), lambda i: (0, 0)),
            pl.BlockSpec((1, _D), lambda i: (0, 0)),
            pl.BlockSpec((1, _D), lambda i: (0, 0)),
        ],
        out_specs=pl.BlockSpec((_ROW_BLK, _D), lambda i: (i, 0)),
        out_shape=jax.ShapeDtypeStruct((_N, _D), jnp.float32),
    )(x, p, W, b.reshape(1, _D), g.reshape(1, _D), be.reshape(1, _D))


def _final_body(x_ref, batch_ref, f1w_ref, f1b_ref, f2w_ref, f2b_ref, o_ref):
    gid = lax.broadcasted_iota(jnp.int32, (_G, _N), 0)
    onehot = (batch_ref[...] == gid).astype(jnp.float32)
    cnt = jnp.sum(onehot, axis=1, keepdims=True)
    sums = jnp.dot(onehot, x_ref[...], preferred_element_type=jnp.float32,
                   precision=lax.Precision.HIGHEST)
    pooled = sums / jnp.maximum(cnt, 1.0)
    z = jnp.dot(pooled, f1w_ref[...], preferred_element_type=jnp.float32)
    z = z + f1b_ref[...]
    z = jnp.where(z > 0.0, z, jnp.exp(jnp.minimum(z, 0.0)) - 1.0)
    z = jnp.dot(z, f2w_ref[...], preferred_element_type=jnp.float32)
    z = z + f2b_ref[...]
    m = jnp.max(z, axis=0, keepdims=True)
    lse = jnp.log(jnp.sum(jnp.exp(z - m), axis=0, keepdims=True))
    o_ref[...] = z - m - lse


def _final(x, batch, fc1W, fc1b, fc2W, fc2b):
    return pl.pallas_call(
        _final_body,
        out_shape=jax.ShapeDtypeStruct((_G, _C), jnp.float32),
    )(x, batch.reshape(1, _N), fc1W, fc1b.reshape(1, _D),
      fc2W, fc2b.reshape(1, _C))


def kernel(h, edge_index, edge_attr, batch,
           W0, b0, g0, be0, W1, b1, g1, be1,
           W2, b2, g2, be2, W3, b3, g3, be3,
           fc1W, fc1b, fc2W, fc2b):
    del edge_attr
    zeros = jnp.zeros((_N, _D), jnp.float32)
    perm = jnp.argsort(edge_index[1], stable=True)
    src = edge_index[0][perm]
    dst = edge_index[1][perm]
    x = h
    for (W, b, g, be) in ((W0, b0, g0, be0), (W1, b1, g1, be1),
                          (W2, b2, g2, be2), (W3, b3, g3, be3)):
        p = _edge_agg(x, src, dst, zeros)
        x = _layer(x, p, W, b, g, be)
    return _final(x, batch, fc1W, fc1b, fc2W, fc2b)
